# baseline (speedup 1.0000x reference)
import functools

import jax
import jax.numpy as jnp
from jax import lax
from jax.experimental import pallas as pl
from jax.experimental.pallas import tpu as pltpu
from jax.experimental.pallas import tpu_sc as plsc

NC = 2
NS = 16
LANES = 16
K = 128
ST = 8
NB0 = 56
RP = 624


def _spmm_sc(n, d, nblk):
    tail = n - NS * RP
    mesh = plsc.VectorSubcoreMesh(core_axis_name="c", subcore_axis_name="s")

    @functools.partial(
        pl.kernel,
        out_type=jax.ShapeDtypeStruct((NC, n, d), jnp.float32),
        mesh=mesh,
        scratch_types=[
            pltpu.VMEM_SHARED((n, d), jnp.float32),
            pltpu.VMEM((ST, K), jnp.int32),
            pltpu.VMEM((ST, K), jnp.int32),
            pltpu.VMEM((ST, K), jnp.float32),
            pltpu.VMEM((K, d), jnp.float32),
        ],
    )
    def spmm(x_hbm, src_hbm, dst_hbm, w_hbm, out_hbm, acc, srcv, dstv, wv,
             rows):
        cid = lax.axis_index("c")
        sid = lax.axis_index("s")

        @pl.loop(0, K)
        def _(r):
            for c in range(d // LANES):
                rows[r, pl.ds(c * LANES, LANES)] = jnp.zeros((LANES,), jnp.float32)

        @pl.loop(0, RP // K)
        def _(z):
            pltpu.sync_copy(rows, acc.at[pl.ds(sid * RP + z * K, K)])

        rem = RP - (RP // K) * K
        if rem:
            pltpu.sync_copy(rows.at[pl.ds(0, rem)],
                            acc.at[pl.ds(sid * RP + RP - rem, rem)])

        @pl.when(sid == 0)
        def _():
            pltpu.sync_copy(rows.at[pl.ds(0, tail)], acc.at[pl.ds(NS * RP, tail)])

        plsc.subcore_barrier()

        def run(base, nstage):
            @pl.loop(0, nstage)
            def _(t):
                off = base + t * ST
                pltpu.sync_copy(src_hbm.at[sid, pl.ds(off, ST)], srcv)
                pltpu.sync_copy(dst_hbm.at[sid, pl.ds(off, ST)], dstv)
                pltpu.sync_copy(w_hbm.at[sid, pl.ds(off, ST)], wv)

                @pl.loop(0, ST)
                def _(b):
                    pltpu.sync_copy(x_hbm.at[srcv.at[b]], rows)

                    @pl.loop(0, K, step=LANES)
                    def _(i0):
                        w16 = wv[b, pl.ds(i0, LANES)]
                        for j in range(LANES):
                            wj = w16[j]
                            for c in range(d // LANES):
                                sl = (i0 + j, pl.ds(c * LANES, LANES))
                                rows[sl] = rows[sl] * wj

                    pltpu.sync_copy(rows, acc.at[dstv.at[b]], add=True)

        @pl.when(cid == 0)
        def _():
            run(0, NB0 // ST)

        @pl.when(cid == 1)
        def _():
            run(NB0, (nblk - NB0) // ST)

        plsc.subcore_barrier()
        pltpu.sync_copy(acc.at[pl.ds(sid * RP, RP)],
                        out_hbm.at[cid, pl.ds(sid * RP, RP)])

        @pl.when(sid == 0)
        def _():
            pltpu.sync_copy(acc.at[pl.ds(NS * RP, tail)],
                            out_hbm.at[cid, pl.ds(NS * RP, tail)])

    return spmm


def _gate_tc(x, W1, b1, W2, b2):
    n, d = x.shape
    h = W1.shape[1]
    bn = 1000

    def body(x_ref, w1_ref, b1_ref, w2_ref, b2_ref, o_ref):
        hid = jnp.maximum(
            jnp.dot(x_ref[...], w1_ref[...],
                    preferred_element_type=jnp.float32) + b1_ref[...], 0.0)
        a = jnp.dot(hid, w2_ref[...],
                    preferred_element_type=jnp.float32) + b2_ref[...]
        o_ref[...] = jax.nn.sigmoid(a)

    return pl.pallas_call(
        body,
        grid=(n // bn,),
        in_specs=[
            pl.BlockSpec((bn, d), lambda i: (i, 0)),
            pl.BlockSpec((d, h), lambda i: (0, 0)),
            pl.BlockSpec((1, h), lambda i: (0, 0)),
            pl.BlockSpec((h, 1), lambda i: (0, 0)),
            pl.BlockSpec((1, 1), lambda i: (0, 0)),
        ],
        out_specs=pl.BlockSpec((bn, 1), lambda i: (i, 0)),
        out_shape=jax.ShapeDtypeStruct((n, 1), jnp.float32),
    )(x, W1.reshape(d, h), b1.reshape(1, h), W2.reshape(h, 1),
      b2.reshape(1, 1))


def _merge_tc(p):
    _, n, d = p.shape
    bn = 1000

    def body(p_ref, o_ref):
        o_ref[...] = p_ref[0] + p_ref[1]

    return pl.pallas_call(
        body,
        grid=(n // bn,),
        in_specs=[pl.BlockSpec((2, bn, d), lambda i: (0, i, 0))],
        out_specs=pl.BlockSpec((bn, d), lambda i: (i, 0)),
        out_shape=jax.ShapeDtypeStruct((n, d), jnp.float32),
    )(p)


def _final_tc(q, x, alph):
    _, n, d = q.shape
    bn = 1000

    def body(q_ref, x_ref, a_ref, o_ref):
        ax = (q_ref[0] + q_ref[1]) * a_ref[...]
        o_ref[...] = jnp.clip(ax - x_ref[...], -10.0, 10.0)

    return pl.pallas_call(
        body,
        grid=(n // bn,),
        in_specs=[
            pl.BlockSpec((2, bn, d), lambda i: (0, i, 0)),
            pl.BlockSpec((bn, d), lambda i: (i, 0)),
            pl.BlockSpec((bn, 1), lambda i: (i, 0)),
        ],
        out_specs=pl.BlockSpec((bn, d), lambda i: (i, 0)),
        out_shape=jax.ShapeDtypeStruct((n, d), jnp.float32),
    )(q, x, alph)


def kernel(t, x, edge_index, edge_weight, W1, b1, W2, b2):
    n, d = x.shape
    e = edge_index.shape[1]
    nblk = -(-(-(-e // (NS * K))) // ST) * ST
    pad = NS * nblk * K - e

    zi = jnp.zeros((pad,), jnp.int32)
    src = jnp.concatenate([edge_index[0], zi]).reshape(NS, nblk, K)
    dst = jnp.concatenate([edge_index[1], zi]).reshape(NS, nblk, K)
    w = jnp.concatenate([edge_weight,
                         jnp.zeros((pad,), jnp.float32)]).reshape(NS, nblk, K)

    spmm = _spmm_sc(n, d, nblk)
    alph = _gate_tc(x, W1, b1, W2, b2)
    p = spmm(x, src, dst, w)
    ax = _merge_tc(p)
    q = spmm(ax, src, dst, w)
    return _final_tc(q, x, alph)


# speedup vs baseline: 1.6715x; 1.6715x over previous
import functools

import jax
import jax.numpy as jnp
from jax import lax
from jax.experimental import pallas as pl
from jax.experimental.pallas import tpu as pltpu
from jax.experimental.pallas import tpu_sc as plsc

NC = 2
NS = 16
NW = NC * NS
LANES = 16
K = 128
RP = 624


def _spmm_sc(n, d, nblk):
    tail = n - NS * RP
    mesh = plsc.VectorSubcoreMesh(core_axis_name="c", subcore_axis_name="s")

    @functools.partial(
        pl.kernel,
        out_type=jax.ShapeDtypeStruct((NC, n, d), jnp.float32),
        mesh=mesh,
        scratch_types=[
            pltpu.VMEM_SHARED((n, d), jnp.float32),
            pltpu.VMEM((nblk, K), jnp.int32),
            pltpu.VMEM((nblk, K), jnp.int32),
            pltpu.VMEM((nblk, K), jnp.float32),
            pltpu.VMEM((K, d), jnp.float32),
        ],
    )
    def spmm(x_hbm, src_hbm, dst_hbm, w_hbm, out_hbm, acc, srcv, dstv, wv,
             rows):
        cid = lax.axis_index("c")
        sid = lax.axis_index("s")
        wid = cid * NS + sid

        pltpu.sync_copy(src_hbm.at[wid], srcv)
        pltpu.sync_copy(dst_hbm.at[wid], dstv)
        pltpu.sync_copy(w_hbm.at[wid], wv)

        @pl.loop(0, K)
        def _(r):
            for c in range(d // LANES):
                rows[r, pl.ds(c * LANES, LANES)] = jnp.zeros((LANES,), jnp.float32)

        @pl.loop(0, RP // K)
        def _(z):
            pltpu.sync_copy(rows, acc.at[pl.ds(sid * RP + z * K, K)])

        rem = RP - (RP // K) * K
        if rem:
            pltpu.sync_copy(rows.at[pl.ds(0, rem)],
                            acc.at[pl.ds(sid * RP + RP - rem, rem)])

        @pl.when(sid == 0)
        def _():
            pltpu.sync_copy(rows.at[pl.ds(0, tail)], acc.at[pl.ds(NS * RP, tail)])

        plsc.subcore_barrier()

        @pl.loop(0, nblk)
        def _(b):
            pltpu.sync_copy(x_hbm.at[srcv.at[b]], rows)

            @pl.loop(0, K, step=LANES)
            def _(i0):
                w16 = wv[b, pl.ds(i0, LANES)]
                for j in range(LANES):
                    wj = w16[j]
                    for c in range(d // LANES):
                        sl = (i0 + j, pl.ds(c * LANES, LANES))
                        rows[sl] = rows[sl] * wj

            pltpu.sync_copy(rows, acc.at[dstv.at[b]], add=True)

        plsc.subcore_barrier()
        pltpu.sync_copy(acc.at[pl.ds(sid * RP, RP)],
                        out_hbm.at[cid, pl.ds(sid * RP, RP)])

        @pl.when(sid == 0)
        def _():
            pltpu.sync_copy(acc.at[pl.ds(NS * RP, tail)],
                            out_hbm.at[cid, pl.ds(NS * RP, tail)])

    return spmm


def _gate_tc(x, W1, b1, W2, b2):
    n, d = x.shape
    h = W1.shape[1]
    bn = 1000

    def body(x_ref, w1_ref, b1_ref, w2_ref, b2_ref, o_ref):
        hid = jnp.maximum(
            jnp.dot(x_ref[...], w1_ref[...],
                    preferred_element_type=jnp.float32) + b1_ref[...], 0.0)
        a = jnp.dot(hid, w2_ref[...],
                    preferred_element_type=jnp.float32) + b2_ref[...]
        o_ref[...] = jax.nn.sigmoid(a)

    return pl.pallas_call(
        body,
        grid=(n // bn,),
        in_specs=[
            pl.BlockSpec((bn, d), lambda i: (i, 0)),
            pl.BlockSpec((d, h), lambda i: (0, 0)),
            pl.BlockSpec((1, h), lambda i: (0, 0)),
            pl.BlockSpec((h, 1), lambda i: (0, 0)),
            pl.BlockSpec((1, 1), lambda i: (0, 0)),
        ],
        out_specs=pl.BlockSpec((bn, 1), lambda i: (i, 0)),
        out_shape=jax.ShapeDtypeStruct((n, 1), jnp.float32),
    )(x, W1.reshape(d, h), b1.reshape(1, h), W2.reshape(h, 1),
      b2.reshape(1, 1))


def _merge_tc(p):
    _, n, d = p.shape
    bn = 1000

    def body(p_ref, o_ref):
        o_ref[...] = p_ref[0] + p_ref[1]

    return pl.pallas_call(
        body,
        grid=(n // bn,),
        in_specs=[pl.BlockSpec((2, bn, d), lambda i: (0, i, 0))],
        out_specs=pl.BlockSpec((bn, d), lambda i: (i, 0)),
        out_shape=jax.ShapeDtypeStruct((n, d), jnp.float32),
    )(p)


def _final_tc(q, x, alph):
    _, n, d = q.shape
    bn = 1000

    def body(q_ref, x_ref, a_ref, o_ref):
        ax = (q_ref[0] + q_ref[1]) * a_ref[...]
        o_ref[...] = jnp.clip(ax - x_ref[...], -10.0, 10.0)

    return pl.pallas_call(
        body,
        grid=(n // bn,),
        in_specs=[
            pl.BlockSpec((2, bn, d), lambda i: (0, i, 0)),
            pl.BlockSpec((bn, d), lambda i: (i, 0)),
            pl.BlockSpec((bn, 1), lambda i: (i, 0)),
        ],
        out_specs=pl.BlockSpec((bn, d), lambda i: (i, 0)),
        out_shape=jax.ShapeDtypeStruct((n, d), jnp.float32),
    )(q, x, alph)


def kernel(t, x, edge_index, edge_weight, W1, b1, W2, b2):
    n, d = x.shape
    e = edge_index.shape[1]
    nblk = -(-e // (NW * K))
    pad = NW * nblk * K - e

    zi = jnp.zeros((pad,), jnp.int32)
    src = jnp.concatenate([edge_index[0], zi]).reshape(NW, nblk, K)
    dst = jnp.concatenate([edge_index[1], zi]).reshape(NW, nblk, K)
    w = jnp.concatenate([edge_weight,
                         jnp.zeros((pad,), jnp.float32)]).reshape(NW, nblk, K)

    spmm = _spmm_sc(n, d, nblk)
    alph = _gate_tc(x, W1, b1, W2, b2)
    p = spmm(x, src, dst, w)
    ax = _merge_tc(p)
    q = spmm(ax, src, dst, w)
    return _final_tc(q, x, alph)
